# baseline (speedup 1.0000x reference)
import functools
import math

import jax
import jax.numpy as jnp
from jax.experimental import pallas as pl
from jax.experimental.pallas import tpu as pltpu

_T = 1.3
_QHAT = 0.9
_KREG = 5
_LAMDA = 0.01
_CA = 2048
_CB = 512
_NBISECT = 8


def _select_body(x_ref, sizes_ref, vkey_ref, rallow_ref, mrow_ref,
                 m_s, sinv_s, lo_s, macc_s, sacc_s, cacc_s, sumacc_s):
    p = pl.program_id(0)
    b = pl.program_id(1)
    nb = pl.num_programs(1)
    y = x_ref[...] / _T
    bsz = y.shape[0]
    nch = y.shape[1] // 128
    y3 = y.reshape(bsz, nch, 128)

    @pl.when(p == 0)
    def _():
        @pl.when(b == 0)
        def _():
            macc_s[...] = jnp.full(macc_s.shape, -jnp.inf, jnp.float32)
            sacc_s[...] = jnp.zeros(sacc_s.shape, jnp.float32)
        bm = jnp.max(y3, axis=1)
        mn = jnp.maximum(macc_s[...], bm)
        e3 = jnp.exp(y3 - mn[:, None, :])
        sacc_s[...] = sacc_s[...] * jnp.exp(macc_s[...] - mn) + jnp.sum(
            e3, axis=1)
        macc_s[...] = mn

        @pl.when(b == nb - 1)
        def _():
            m = jnp.max(macc_s[...], axis=1, keepdims=True)
            s = jnp.sum(sacc_s[...] * jnp.exp(macc_s[...] - m), axis=1,
                        keepdims=True)
            m_s[...] = m
            sinv_s[...] = 1.0 / s

    @pl.when((p >= 1) & (p <= _NBISECT))
    def _():
        @pl.when(b == 0)
        def _():
            cacc_s[...] = jnp.zeros(cacc_s.shape, jnp.float32)
            sumacc_s[...] = jnp.zeros(sumacc_s.shape, jnp.float32)

            @pl.when(p == 1)
            def _():
                lo_s[...] = jnp.zeros(lo_s.shape, jnp.int32)

        e3 = jnp.exp(y3 - m_s[...][:, :, None])
        key3 = jax.lax.bitcast_convert_type(e3, jnp.int32)
        shift = 30 - 4 * p
        shift = jnp.maximum(shift, 0)
        step = jnp.left_shift(jnp.int32(1), shift)
        lo = lo_s[...]
        d3 = key3 - lo[:, :, None]
        for j in range(1, 16):
            msk = d3 >= j * step
            cacc_s[:, (j - 1) * 128:j * 128] += jnp.sum(
                jnp.where(msk, 1.0, 0.0), axis=1)
            sumacc_s[:, (j - 1) * 128:j * 128] += jnp.sum(
                jnp.where(msk, e3, 0.0), axis=1)

        @pl.when(b == nb - 1)
        def _():
            cl = jnp.concatenate(
                [jnp.sum(cacc_s[:, (j - 1) * 128:j * 128], axis=1,
                         keepdims=True) for j in range(1, 16)], axis=1)
            sl = jnp.concatenate(
                [jnp.sum(sumacc_s[:, (j - 1) * 128:j * 128], axis=1,
                         keepdims=True) for j in range(1, 16)], axis=1)
            fval = sl * sinv_s[...] + _LAMDA * jnp.maximum(cl - _KREG, 0.0)
            jstar = jnp.sum(jnp.where(fval > _QHAT, 1.0, 0.0), axis=1,
                            keepdims=True).astype(jnp.int32)
            lo_s[...] = lo + jstar * step

    @pl.when(p == 1 + _NBISECT)
    def _():
        @pl.when(b == 0)
        def _():
            cacc_s[...] = jnp.zeros(cacc_s.shape, jnp.float32)
            sumacc_s[...] = jnp.zeros(sumacc_s.shape, jnp.float32)

        e3 = jnp.exp(y3 - m_s[...][:, :, None])
        key3 = jax.lax.bitcast_convert_type(e3, jnp.int32)
        vk = lo_s[...]
        vk3 = vk[:, :, None]
        gt = key3 > vk3
        eq = key3 == vk3
        cacc_s[:, 0:128] += jnp.sum(jnp.where(gt, 1.0, 0.0), axis=1)
        cacc_s[:, 128:256] += jnp.sum(jnp.where(eq, 1.0, 0.0), axis=1)
        sumacc_s[:, 0:128] += jnp.sum(jnp.where(gt, e3, 0.0), axis=1)

        @pl.when(b == nb - 1)
        def _():
            sinv = sinv_s[...]
            cnt_gt = jnp.sum(cacc_s[:, 0:128], axis=1, keepdims=True)
            cnt_eq = jnp.sum(cacc_s[:, 128:256], axis=1, keepdims=True)
            sum_gt = jnp.sum(sumacc_s[:, 0:128], axis=1, keepdims=True) * sinv
            sv = jax.lax.bitcast_convert_type(vk, jnp.float32) * sinv
            jj = jax.lax.broadcasted_iota(
                jnp.int32, (bsz, 128), 1).astype(jnp.float32) + 1.0
            mpos = cnt_gt + jj
            fj = sum_gt + jj * sv + _LAMDA * jnp.maximum(mpos - _KREG, 0.0)
            hold = jnp.logical_and(jj <= cnt_eq, fj <= _QHAT)
            qc = jnp.sum(jnp.where(hold, 1.0, 0.0), axis=1, keepdims=True)
            sizes_ref[...] = (cnt_gt + qc + 1.0).astype(jnp.int32)
            rallow_ref[...] = (qc + 1.0).astype(jnp.int32)
            vkey_ref[...] = vk
            mrow_ref[...] = m_s[...]


def _mask_body(x_ref, mrow_ref, vkey_ref, rallow_ref, out_ref, carry_s, lt_s):
    b = pl.program_id(0)

    @pl.when(b == 0)
    def _():
        carry_s[...] = jnp.zeros(carry_s.shape, jnp.float32)
        r = jax.lax.broadcasted_iota(jnp.int32, lt_s.shape, 0)
        c = jax.lax.broadcasted_iota(jnp.int32, lt_s.shape, 1)
        lt_s[...] = jnp.where(r < c, 1.0, 0.0)

    y = x_ref[...] / _T
    e = jnp.exp(y - mrow_ref[...])
    key = jax.lax.bitcast_convert_type(e, jnp.int32)
    vk = vkey_ref[...]
    gt = key > vk
    eq = key == vk
    eqf = jnp.where(eq, 1.0, 0.0)
    rank = carry_s[...] + jnp.dot(eqf, lt_s[...],
                                  preferred_element_type=jnp.float32)
    ra = rallow_ref[...].astype(jnp.float32)
    out_ref[...] = jnp.logical_or(gt, jnp.logical_and(eq, rank < ra))
    carry_s[...] = carry_s[...] + jnp.sum(eqf, axis=1, keepdims=True)


@functools.partial(jax.jit, static_argnames=("interpret",))
def kernel(logits, interpret=False):
    bsz, n = logits.shape
    npad = math.lcm(_CA, _CB) * math.ceil(n / math.lcm(_CA, _CB))
    nba = npad // _CA
    nbb = npad // _CB
    xp = logits
    if npad != n:
        xp = jnp.pad(logits, ((0, 0), (0, npad - n)),
                     constant_values=-jnp.inf)

    npasses = 2 + _NBISECT
    row = functools.partial(pl.BlockSpec, (bsz, 1))
    sizes, vkey, rallow, mrow = pl.pallas_call(
        _select_body,
        grid=(npasses, nba),
        in_specs=[pl.BlockSpec((bsz, _CA), lambda p, b: (0, b))],
        out_specs=[row(lambda p, b: (0, 0)) for _ in range(4)],
        out_shape=[
            jax.ShapeDtypeStruct((bsz, 1), jnp.int32),
            jax.ShapeDtypeStruct((bsz, 1), jnp.int32),
            jax.ShapeDtypeStruct((bsz, 1), jnp.int32),
            jax.ShapeDtypeStruct((bsz, 1), jnp.float32),
        ],
        scratch_shapes=[
            pltpu.VMEM((bsz, 1), jnp.float32),
            pltpu.VMEM((bsz, 1), jnp.float32),
            pltpu.VMEM((bsz, 1), jnp.int32),
            pltpu.VMEM((bsz, 128), jnp.float32),
            pltpu.VMEM((bsz, 128), jnp.float32),
            pltpu.VMEM((bsz, 15 * 128), jnp.float32),
            pltpu.VMEM((bsz, 15 * 128), jnp.float32),
        ],
        interpret=interpret,
    )(xp)

    mask = pl.pallas_call(
        _mask_body,
        grid=(nbb,),
        in_specs=[
            pl.BlockSpec((bsz, _CB), lambda b: (0, b)),
            pl.BlockSpec((bsz, 1), lambda b: (0, 0)),
            pl.BlockSpec((bsz, 1), lambda b: (0, 0)),
            pl.BlockSpec((bsz, 1), lambda b: (0, 0)),
        ],
        out_specs=pl.BlockSpec((bsz, _CB), lambda b: (0, b)),
        out_shape=jax.ShapeDtypeStruct((bsz, npad), jnp.bool_),
        scratch_shapes=[
            pltpu.VMEM((bsz, 1), jnp.float32),
            pltpu.VMEM((_CB, _CB), jnp.float32),
        ],
        interpret=interpret,
    )(xp, mrow, vkey, rallow)

    return (logits, mask[:, :n], sizes.reshape(bsz))


# speedup vs baseline: 1.3237x; 1.3237x over previous
import functools
import math

import jax
import jax.numpy as jnp
from jax.experimental import pallas as pl
from jax.experimental.pallas import tpu as pltpu

_T = 1.3
_QHAT = 0.9
_KREG = 5
_LAMDA = 0.01
_CA = 2048
_CB = 512
_NBISECT = 8


def _select_body(x_ref, sizes_ref, vkey_ref, rallow_ref, mrow_ref,
                 m_s, s_s, sinv_s, lo_s, cnt_s, sum_s):
    p = pl.program_id(1)
    b = pl.program_id(2)
    nb = pl.num_programs(2)
    y = x_ref[...] / _T

    @pl.when(p == 0)
    def _():
        @pl.when(b == 0)
        def _():
            m_s[...] = jnp.full(m_s.shape, -jnp.inf, jnp.float32)
            s_s[...] = jnp.zeros(s_s.shape, jnp.float32)
        m_old = m_s[...]
        m_new = jnp.maximum(m_old, jnp.max(y, axis=1, keepdims=True))
        e = jnp.exp(y - m_new)
        s_s[...] = s_s[...] * jnp.exp(m_old - m_new) + jnp.sum(
            e, axis=1, keepdims=True)
        m_s[...] = m_new

        @pl.when(b == nb - 1)
        def _():
            sinv_s[...] = 1.0 / s_s[...]

    @pl.when((p >= 1) & (p <= _NBISECT))
    def _():
        @pl.when(b == 0)
        def _():
            cnt_s[...] = jnp.zeros(cnt_s.shape, jnp.float32)
            sum_s[...] = jnp.zeros(sum_s.shape, jnp.float32)

            @pl.when(p == 1)
            def _():
                lo_s[...] = jnp.zeros(lo_s.shape, jnp.int32)

        e = jnp.exp(y - m_s[...])
        key = jax.lax.bitcast_convert_type(e, jnp.int32)
        shift = 30 - 4 * p
        shift = jnp.maximum(shift, 0)
        step = jnp.left_shift(jnp.int32(1), shift)
        lo = lo_s[...]
        cparts = []
        sparts = []
        for j in range(1, 16):
            msk = key >= lo + j * step
            cparts.append(jnp.sum(jnp.where(msk, 1.0, 0.0), axis=1,
                                  keepdims=True))
            sparts.append(jnp.sum(jnp.where(msk, e, 0.0), axis=1,
                                  keepdims=True))
        cnt_s[:, 1:16] += jnp.concatenate(cparts, axis=1)
        sum_s[:, 1:16] += jnp.concatenate(sparts, axis=1)

        @pl.when(b == nb - 1)
        def _():
            cnt = cnt_s[...]
            fval = sum_s[...] * sinv_s[...] + _LAMDA * jnp.maximum(
                cnt - _KREG, 0.0)
            colj = jax.lax.broadcasted_iota(jnp.int32, fval.shape, 1)
            pred = jnp.logical_or(fval > _QHAT, colj == 0)
            jstar = (jnp.sum(jnp.where(pred, 1.0, 0.0), axis=1,
                             keepdims=True) - 1.0).astype(jnp.int32)
            lo_s[...] = lo + jstar * step

    @pl.when(p == 1 + _NBISECT)
    def _():
        @pl.when(b == 0)
        def _():
            cnt_s[...] = jnp.zeros(cnt_s.shape, jnp.float32)
            sum_s[...] = jnp.zeros(sum_s.shape, jnp.float32)

        e = jnp.exp(y - m_s[...])
        key = jax.lax.bitcast_convert_type(e, jnp.int32)
        vk = lo_s[...]
        gt = key > vk
        eq = key == vk
        cnt_s[:, 0:1] += jnp.sum(jnp.where(gt, 1.0, 0.0), axis=1,
                                 keepdims=True)
        cnt_s[:, 1:2] += jnp.sum(jnp.where(eq, 1.0, 0.0), axis=1,
                                 keepdims=True)
        sum_s[:, 0:1] += jnp.sum(jnp.where(gt, e, 0.0), axis=1,
                                 keepdims=True)

        @pl.when(b == nb - 1)
        def _():
            sinv = sinv_s[...]
            cnt_gt = cnt_s[:, 0:1]
            cnt_eq = cnt_s[:, 1:2]
            sum_gt = sum_s[:, 0:1] * sinv
            sv = jax.lax.bitcast_convert_type(vk, jnp.float32) * sinv
            bsz = vk.shape[0]
            jj = jax.lax.broadcasted_iota(
                jnp.int32, (bsz, 128), 1).astype(jnp.float32) + 1.0
            mpos = cnt_gt + jj
            fj = sum_gt + jj * sv + _LAMDA * jnp.maximum(mpos - _KREG, 0.0)
            hold = jnp.logical_and(jj <= cnt_eq, fj <= _QHAT)
            qc = jnp.sum(jnp.where(hold, 1.0, 0.0), axis=1, keepdims=True)
            sizes_ref[...] = (cnt_gt + qc + 1.0).astype(jnp.int32)
            rallow_ref[...] = (qc + 1.0).astype(jnp.int32)
            vkey_ref[...] = vk
            mrow_ref[...] = m_s[...]


def _mask_body(x_ref, mrow_ref, vkey_ref, rallow_ref, out_ref, carry_s, lt_s):
    b = pl.program_id(1)

    @pl.when(b == 0)
    def _():
        carry_s[...] = jnp.zeros(carry_s.shape, jnp.float32)
        r = jax.lax.broadcasted_iota(jnp.int32, lt_s.shape, 0)
        c = jax.lax.broadcasted_iota(jnp.int32, lt_s.shape, 1)
        lt_s[...] = jnp.where(r < c, 1.0, 0.0)

    y = x_ref[...] / _T
    e = jnp.exp(y - mrow_ref[...])
    key = jax.lax.bitcast_convert_type(e, jnp.int32)
    vk = vkey_ref[...]
    gt = key > vk
    eq = key == vk
    eqf = jnp.where(eq, 1.0, 0.0)
    rank = carry_s[...] + jnp.dot(eqf, lt_s[...],
                                  preferred_element_type=jnp.float32)
    ra = rallow_ref[...].astype(jnp.float32)
    out_ref[...] = jnp.logical_or(gt, jnp.logical_and(eq, rank < ra))
    carry_s[...] = carry_s[...] + jnp.sum(eqf, axis=1, keepdims=True)


@functools.partial(jax.jit, static_argnames=("interpret",))
def kernel(logits, interpret=False):
    bsz, n = logits.shape
    npad = math.lcm(_CA, _CB) * math.ceil(n / math.lcm(_CA, _CB))
    nba = npad // _CA
    nbb = npad // _CB
    xp = logits
    if npad != n:
        xp = jnp.pad(logits, ((0, 0), (0, npad - n)),
                     constant_values=-jnp.inf)

    rs = 2 if (bsz // 2) % 8 == 0 else 1
    rb = bsz // rs
    npasses = 2 + _NBISECT
    row = functools.partial(pl.BlockSpec, (rb, 1))
    sizes, vkey, rallow, mrow = pl.pallas_call(
        _select_body,
        grid=(rs, npasses, nba),
        in_specs=[pl.BlockSpec((rb, _CA), lambda r, p, b: (r, b))],
        out_specs=[row(lambda r, p, b: (r, 0)) for _ in range(4)],
        out_shape=[
            jax.ShapeDtypeStruct((bsz, 1), jnp.int32),
            jax.ShapeDtypeStruct((bsz, 1), jnp.int32),
            jax.ShapeDtypeStruct((bsz, 1), jnp.int32),
            jax.ShapeDtypeStruct((bsz, 1), jnp.float32),
        ],
        scratch_shapes=[
            pltpu.VMEM((rb, 1), jnp.float32),
            pltpu.VMEM((rb, 1), jnp.float32),
            pltpu.VMEM((rb, 1), jnp.float32),
            pltpu.VMEM((rb, 1), jnp.int32),
            pltpu.VMEM((rb, 16), jnp.float32),
            pltpu.VMEM((rb, 16), jnp.float32),
        ],
        compiler_params=pltpu.CompilerParams(
            dimension_semantics=("parallel", "arbitrary", "arbitrary")),
        interpret=interpret,
    )(xp)

    mask = pl.pallas_call(
        _mask_body,
        grid=(rs, nbb),
        in_specs=[
            pl.BlockSpec((rb, _CB), lambda r, b: (r, b)),
            pl.BlockSpec((rb, 1), lambda r, b: (r, 0)),
            pl.BlockSpec((rb, 1), lambda r, b: (r, 0)),
            pl.BlockSpec((rb, 1), lambda r, b: (r, 0)),
        ],
        out_specs=pl.BlockSpec((rb, _CB), lambda r, b: (r, b)),
        out_shape=jax.ShapeDtypeStruct((bsz, npad), jnp.bool_),
        scratch_shapes=[
            pltpu.VMEM((rb, 1), jnp.float32),
            pltpu.VMEM((_CB, _CB), jnp.float32),
        ],
        compiler_params=pltpu.CompilerParams(
            dimension_semantics=("parallel", "arbitrary")),
        interpret=interpret,
    )(xp, mrow, vkey, rallow)

    return (logits, mask[:, :n], sizes.reshape(bsz))
